# baseline (speedup 1.0000x reference)
import dataclasses
import functools

import jax
import jax.numpy as jnp
from jax import lax
from jax.experimental import pallas as pl
from jax.experimental.pallas import tpu as pltpu
from jax.experimental.pallas import tpu_sc as plsc

NC = 2
NS = 16
NW = NC * NS
LANES = 16


def _sc_compiler_params():
    cp = pltpu.CompilerParams()
    if "needs_layout_passes" in pltpu.CompilerParams.__dataclass_fields__:
        cp = dataclasses.replace(cp, needs_layout_passes=False)
    return cp


def _sc_degree(col, attr, n_pad, nchunk):
    chunk = CHUNK
    epw = nchunk * chunk
    rows_per_tile = n_pad // NS
    zrows = rows_per_tile // 5
    deg_w = 128

    mesh = plsc.VectorSubcoreMesh(core_axis_name="c", subcore_axis_name="s")

    @functools.partial(
        pl.kernel,
        out_type=jax.ShapeDtypeStruct((NC, n_pad, deg_w), jnp.float32),
        mesh=mesh,
        scratch_types=[
            pltpu.VMEM((1, chunk), jnp.int32),
            pltpu.VMEM((1, chunk), jnp.int32),
            pltpu.VMEM((chunk,), jnp.float32),
            pltpu.VMEM((chunk,), jnp.float32),
            pltpu.VMEM((chunk, deg_w), jnp.float32),
            pltpu.VMEM((zrows, deg_w), jnp.float32),
            pltpu.VMEM_SHARED((n_pad, deg_w), jnp.float32),
            pltpu.SemaphoreType.DMA,
            pltpu.SemaphoreType.DMA,
        ],
        compiler_params=_sc_compiler_params(),
    )
    def k(col_hbm, attr_hbm, out_hbm, cbuf0, cbuf1, abuf0, abuf1,
          srcv, zbuf, acc, si0, si1):
        c = lax.axis_index("c")
        s = lax.axis_index("s")
        wid = s * NC + c
        base = wid * epw

        cbuf = (cbuf0, cbuf1)
        abuf = (abuf0, abuf1)
        si = (si0, si1)

        def issue_idx(i, b):
            off = base + i * chunk
            pltpu.async_copy(col_hbm.at[pl.ds(off, chunk)], cbuf[b].at[0], si[b])
            pltpu.async_copy(attr_hbm.at[pl.ds(off, chunk)], abuf[b], si[b])

        def wait_idx(i, b):
            off = base + i * chunk
            pltpu.make_async_copy(col_hbm.at[pl.ds(off, chunk)], cbuf[b].at[0], si[b]).wait()
            pltpu.make_async_copy(attr_hbm.at[pl.ds(off, chunk)], abuf[b], si[b]).wait()

        @pl.loop(0, zrows)
        def _(i):
            for kk in range(deg_w // LANES):
                zbuf[i, pl.ds(kk * LANES, LANES)] = jnp.zeros((LANES,), jnp.float32)

        @pl.loop(0, chunk)
        def _(i):
            for kk in range(deg_w // LANES):
                srcv[i, pl.ds(kk * LANES, LANES)] = jnp.zeros((LANES,), jnp.float32)

        @pl.loop(0, 5)
        def _(t):
            pltpu.sync_copy(zbuf, acc.at[pl.ds(s * rows_per_tile + t * zrows, zrows)])

        plsc.subcore_barrier()

        issue_idx(0, 0)
        issue_idx(1, 1)

        @pl.loop(0, nchunk // 2)
        def _(t):
            for b in range(2):
                i = t * 2 + b
                wait_idx(i, b)
                ab = abuf[b]

                @plsc.parallel_loop(0, chunk, unroll=4)
                def _(j):
                    j16 = jnp.zeros((LANES,), jnp.int32) + j
                    w = plsc.load_gather(ab, [j16])
                    srcv[j, pl.ds(0, LANES)] = w

                pltpu.sync_copy(srcv, acc.at[cbuf[b].at[0]], add=True)

                @pl.when(i + 2 < nchunk)
                def _():
                    issue_idx(i + 2, b)

        plsc.subcore_barrier()

        @pl.loop(0, 5)
        def _(t):
            st = s * rows_per_tile + t * zrows
            pltpu.sync_copy(acc.at[pl.ds(st, zrows)],
                            out_hbm.at[c, pl.ds(st, zrows)])

    return k(col, attr)


CHUNK = 96


def _sc_aggregate(h3, row, col, attr, n_pad, d, nchunk):
    chunk = CHUNK
    epw = nchunk * chunk
    rows_per_tile = n_pad // NS
    zrows = rows_per_tile // 5

    mesh = plsc.VectorSubcoreMesh(core_axis_name="c", subcore_axis_name="s")

    @functools.partial(
        pl.kernel,
        out_type=jax.ShapeDtypeStruct((NC, n_pad, d), jnp.float32),
        mesh=mesh,
        scratch_types=[
            pltpu.VMEM((2, chunk), jnp.int32),
            pltpu.VMEM((2, chunk), jnp.int32),
            pltpu.VMEM((2, chunk), jnp.int32),
            pltpu.VMEM((chunk,), jnp.float32),
            pltpu.VMEM((chunk,), jnp.float32),
            pltpu.VMEM((chunk,), jnp.float32),
            pltpu.VMEM((chunk, d), jnp.float32),
            pltpu.VMEM((chunk, d), jnp.float32),
            pltpu.VMEM((zrows, d), jnp.float32),
            pltpu.VMEM_SHARED((n_pad, d), jnp.float32),
            pltpu.SemaphoreType.DMA,
            pltpu.SemaphoreType.DMA,
            pltpu.SemaphoreType.DMA,
            pltpu.SemaphoreType.DMA,
            pltpu.SemaphoreType.DMA,
            pltpu.SemaphoreType.DMA,
            pltpu.SemaphoreType.DMA,
        ],
        compiler_params=_sc_compiler_params(),
    )
    def k(h3_hbm, row_hbm, col_hbm, attr_hbm, out_hbm,
          ibuf0, ibuf1, ibuf2, abuf0, abuf1, abuf2, gbuf0, gbuf1, zbuf, acc,
          si0, si1, si2, sg0, sg1, ss0, ss1):
        c = lax.axis_index("c")
        s = lax.axis_index("s")
        wid = s * NC + c
        base = wid * epw

        ibuf = (ibuf0, ibuf1, ibuf2)
        abuf = (abuf0, abuf1, abuf2)
        gbuf = (gbuf0, gbuf1)
        si = (si0, si1, si2)
        sg = (sg0, sg1)
        ss = (ss0, ss1)

        def issue_idx(i, b):
            off = base + i * chunk
            pltpu.async_copy(row_hbm.at[pl.ds(off, chunk)], ibuf[b].at[0], si[b])
            pltpu.async_copy(col_hbm.at[pl.ds(off, chunk)], ibuf[b].at[1], si[b])
            pltpu.async_copy(attr_hbm.at[pl.ds(off, chunk)], abuf[b], si[b])

        def wait_idx(i, b):
            off = base + i * chunk
            pltpu.make_async_copy(row_hbm.at[pl.ds(off, chunk)], ibuf[b].at[0], si[b]).wait()
            pltpu.make_async_copy(col_hbm.at[pl.ds(off, chunk)], ibuf[b].at[1], si[b]).wait()
            pltpu.make_async_copy(attr_hbm.at[pl.ds(off, chunk)], abuf[b], si[b]).wait()

        def issue_gather(b3, b2):
            pltpu.async_copy(h3_hbm.at[ibuf[b3].at[0]], gbuf[b2], sg[b2])

        def wait_gather(b3, b2):
            pltpu.make_async_copy(h3_hbm.at[ibuf[b3].at[0]], gbuf[b2], sg[b2]).wait()

        def issue_scatter(b3, b2):
            pltpu.async_copy(gbuf[b2], acc.at[ibuf[b3].at[1]], ss[b2], add=True)

        def wait_scatter(b3, b2):
            pltpu.make_async_copy(gbuf[b2], acc.at[ibuf[b3].at[1]], ss[b2]).wait()

        @pl.loop(0, zrows)
        def _(i):
            for kk in range(d // LANES):
                zbuf[i, pl.ds(kk * LANES, LANES)] = jnp.zeros((LANES,), jnp.float32)

        @pl.loop(0, 5)
        def _(t):
            pltpu.sync_copy(zbuf, acc.at[pl.ds(s * rows_per_tile + t * zrows, zrows)])

        plsc.subcore_barrier()

        issue_idx(0, 0)
        issue_idx(1, 1)
        wait_idx(0, 0)
        issue_gather(0, 0)

        @pl.loop(0, nchunk // 6)
        def _(t):
            for u in range(6):
                b3 = u % 3
                b2 = u % 2
                n3 = (u + 1) % 3
                n2 = (u + 1) % 2
                i = t * 6 + u

                wait_gather(b3, b2)

                gb = gbuf[b2]
                ab = abuf[b3]

                @plsc.parallel_loop(0, chunk, unroll=4)
                def _(j):
                    j16 = jnp.zeros((LANES,), jnp.int32) + j
                    w = plsc.load_gather(ab, [j16])
                    for kk in range(d // LANES):
                        sl = pl.ds(kk * LANES, LANES)
                        gb[j, sl] = gb[j, sl] * w

                issue_scatter(b3, b2)

                @pl.when(i + 1 < nchunk)
                def _():
                    wait_idx(i + 1, n3)

                @pl.when(i >= 1)
                def _():
                    wait_scatter((u + 2) % 3, n2)

                @pl.when(i + 1 < nchunk)
                def _():
                    issue_gather(n3, n2)

                @pl.when(i + 2 < nchunk)
                def _():
                    issue_idx(i + 2, (u + 2) % 3)

        wait_scatter((nchunk - 1) % 3, (nchunk - 1) % 2)

        plsc.subcore_barrier()

        @pl.loop(0, 5)
        def _(t):
            st = s * rows_per_tile + t * zrows
            pltpu.sync_copy(acc.at[pl.ds(st, zrows)],
                            out_hbm.at[c, pl.ds(st, zrows)])

    return k(h3, row, col, attr)


def _bn(x, gamma, beta):
    mean = jnp.mean(x, axis=0)
    xc = x - mean
    var = jnp.mean(xc * xc, axis=0)
    return gamma * xc / jnp.sqrt(var + 1e-5) + beta


def _tc_layer0(x, gamma, beta, w):
    def body(x_ref, g_ref, b_ref, w_ref, o_ref):
        h = _bn(x_ref[...], g_ref[...], b_ref[...])
        o_ref[...] = jnp.dot(h, w_ref[...], preferred_element_type=jnp.float32)

    return pl.pallas_call(
        body,
        out_shape=jax.ShapeDtypeStruct(x.shape, jnp.float32),
    )(x, gamma, beta, w)


def _tc_scale(degp, h2):
    n, d = h2.shape

    def body(degp_ref, h2_ref, dis_ref, h3_ref):
        deg = degp_ref[0, :n, :LANES] + degp_ref[1, :n, :LANES] + 2.0
        dis = jnp.where(deg > 0, lax.rsqrt(deg), 0.0)
        dis_ref[...] = dis
        h3_ref[...] = dis[:, 0:1] * h2_ref[...]

    return pl.pallas_call(
        body,
        out_shape=[
            jax.ShapeDtypeStruct((n, LANES), jnp.float32),
            jax.ShapeDtypeStruct((n, d), jnp.float32),
        ],
    )(degp, h2)


def _tc_combine_next(raw, h2, dis16, b, gamma, beta, w):
    n, d = h2.shape

    def body(raw_ref, h2_ref, dis_ref, b_ref, g_ref, bt_ref, w_ref,
             h2o_ref, h3o_ref):
        d1 = dis_ref[:, 0:1]
        h = (d1 * (raw_ref[0, :n, :] + raw_ref[1, :n, :])
             + (2.0 * d1 * d1) * h2_ref[...] + b_ref[...])
        h = jnp.where(h >= 0, h, 0.01 * h)
        h = _bn(h, g_ref[...], bt_ref[...])
        h2n = jnp.dot(h, w_ref[...], preferred_element_type=jnp.float32)
        h2o_ref[...] = h2n
        h3o_ref[...] = d1 * h2n

    return pl.pallas_call(
        body,
        out_shape=[
            jax.ShapeDtypeStruct((n, d), jnp.float32),
            jax.ShapeDtypeStruct((n, d), jnp.float32),
        ],
    )(raw, h2, dis16, b, gamma, beta, w)


def _tc_finish_pool(raw, h2, dis16, b, batch, g):
    n, d = h2.shape

    def body(raw_ref, h2_ref, dis_ref, b_ref, batch_ref, o_ref):
        d1 = dis_ref[:, 0:1]
        h = (d1 * (raw_ref[0, :n, :] + raw_ref[1, :n, :])
             + (2.0 * d1 * d1) * h2_ref[...] + b_ref[...])
        h = jnp.where(h >= 0, h, 0.01 * h)
        gids = lax.broadcasted_iota(jnp.int32, (g, n), 0)
        oh = (gids == batch_ref[...][None, :]).astype(jnp.float32)
        sums = jnp.dot(oh, h, preferred_element_type=jnp.float32)
        cnt = jnp.sum(oh, axis=1)
        o_ref[...] = sums / jnp.maximum(cnt, 1.0)[:, None]

    return pl.pallas_call(
        body,
        out_shape=jax.ShapeDtypeStruct((g, d), jnp.float32),
    )(raw, h2, dis16, b, batch)


def kernel(x, edge_index, edge_attr, batch,
           bn_gamma0, bn_beta0, W0, b0,
           bn_gamma1, bn_beta1, W1, b1,
           bn_gamma2, bn_beta2, W2, b2):
    n, d = x.shape
    e = edge_index.shape[1]
    g = 64
    n_pad = ((n + NS * 40 - 1) // (NS * 40)) * (NS * 40)
    row = edge_index[0]
    col = edge_index[1]
    ew = 1.0 - edge_attr[:, 0]

    epw0 = e // NW
    nchunk = ((epw0 + CHUNK - 1) // CHUNK + 5) // 6 * 6
    padn = nchunk * CHUNK - epw0
    if padn:
        ar = jnp.arange(padn, dtype=jnp.int32)
        row = jnp.concatenate(
            [row.reshape(NW, epw0),
             jnp.broadcast_to((ar * 131) % n, (NW, padn))], axis=1).reshape(-1)
        col = jnp.concatenate(
            [col.reshape(NW, epw0),
             jnp.broadcast_to(n + ar % (n_pad - n), (NW, padn))], axis=1).reshape(-1)
        ew = jnp.concatenate(
            [ew.reshape(NW, epw0),
             jnp.zeros((NW, padn), jnp.float32)], axis=1).reshape(-1)

    degp = _sc_degree(col, ew, n_pad, nchunk)
    h2 = _tc_layer0(x, bn_gamma0, bn_beta0, W0)
    dis16, h3 = _tc_scale(degp, h2)

    params = [(b0, bn_gamma1, bn_beta1, W1), (b1, bn_gamma2, bn_beta2, W2)]
    for (bl, gm, bt, wl) in params:
        raw = _sc_aggregate(h3, row, col, ew, n_pad, d, nchunk)
        h2, h3 = _tc_combine_next(raw, h2, dis16, bl, gm, bt, wl)

    raw = _sc_aggregate(h3, row, col, ew, n_pad, d, nchunk)
    return _tc_finish_pool(raw, h2, dis16, b2, batch, g)


# speedup vs baseline: 1.1637x; 1.1637x over previous
import dataclasses
import functools

import jax
import jax.numpy as jnp
from jax import lax
from jax.experimental import pallas as pl
from jax.experimental.pallas import tpu as pltpu
from jax.experimental.pallas import tpu_sc as plsc

NC = 2
NS = 16
NW = NC * NS
LANES = 16


def _sc_compiler_params():
    cp = pltpu.CompilerParams()
    if "needs_layout_passes" in pltpu.CompilerParams.__dataclass_fields__:
        cp = dataclasses.replace(cp, needs_layout_passes=False)
    return cp


def _sc_degree(col, attr, n_pad, nchunk):
    chunk = CHUNK
    epw = nchunk * chunk
    rows_per_tile = n_pad // NS
    zrows = rows_per_tile // 5
    deg_w = 128

    mesh = plsc.VectorSubcoreMesh(core_axis_name="c", subcore_axis_name="s")

    @functools.partial(
        pl.kernel,
        out_type=jax.ShapeDtypeStruct((NC, n_pad, deg_w), jnp.float32),
        mesh=mesh,
        scratch_types=[
            pltpu.VMEM((1, chunk), jnp.int32),
            pltpu.VMEM((1, chunk), jnp.int32),
            pltpu.VMEM((chunk,), jnp.float32),
            pltpu.VMEM((chunk,), jnp.float32),
            pltpu.VMEM((chunk, deg_w), jnp.float32),
            pltpu.VMEM_SHARED((n_pad, deg_w), jnp.float32),
            pltpu.SemaphoreType.DMA,
            pltpu.SemaphoreType.DMA,
        ],
        compiler_params=_sc_compiler_params(),
    )
    def k(col_hbm, attr_hbm, out_hbm, cbuf0, cbuf1, abuf0, abuf1,
          srcv, acc, si0, si1):
        c = lax.axis_index("c")
        s = lax.axis_index("s")
        wid = s * NC + c
        base = wid * epw

        cbuf = (cbuf0, cbuf1)
        abuf = (abuf0, abuf1)
        si = (si0, si1)

        def issue_idx(i, b):
            off = base + i * chunk
            pltpu.async_copy(col_hbm.at[pl.ds(off, chunk)], cbuf[b].at[0], si[b])
            pltpu.async_copy(attr_hbm.at[pl.ds(off, chunk)], abuf[b], si[b])

        def wait_idx(i, b):
            off = base + i * chunk
            pltpu.make_async_copy(col_hbm.at[pl.ds(off, chunk)], cbuf[b].at[0], si[b]).wait()
            pltpu.make_async_copy(attr_hbm.at[pl.ds(off, chunk)], abuf[b], si[b]).wait()

        @pl.loop(0, chunk)
        def _(i):
            for kk in range(deg_w // LANES):
                srcv[i, pl.ds(kk * LANES, LANES)] = jnp.zeros((LANES,), jnp.float32)

        @pl.loop(0, 5)
        def _(t):
            pltpu.sync_copy(srcv.at[pl.ds(0, zrows)],
                            acc.at[pl.ds(s * rows_per_tile + t * zrows, zrows)])

        plsc.subcore_barrier()

        issue_idx(0, 0)
        issue_idx(1, 1)

        @pl.loop(0, nchunk // 2)
        def _(t):
            for b in range(2):
                i = t * 2 + b
                wait_idx(i, b)
                ab = abuf[b]

                @plsc.parallel_loop(0, chunk, unroll=4)
                def _(j):
                    j16 = jnp.zeros((LANES,), jnp.int32) + j
                    w = plsc.load_gather(ab, [j16])
                    srcv[j, pl.ds(0, LANES)] = w

                pltpu.sync_copy(srcv, acc.at[cbuf[b].at[0]], add=True)

                @pl.when(i + 2 < nchunk)
                def _():
                    issue_idx(i + 2, b)

        plsc.subcore_barrier()

        @pl.loop(0, 5)
        def _(t):
            st = s * rows_per_tile + t * zrows
            pltpu.sync_copy(acc.at[pl.ds(st, zrows)],
                            out_hbm.at[c, pl.ds(st, zrows)])

    return k(col, attr)


CHUNK = 128


def _sc_aggregate(h3, row, col, attr, n_pad, d, nchunk):
    chunk = CHUNK
    epw = nchunk * chunk
    rows_per_tile = n_pad // NS
    zrows = rows_per_tile // 5

    mesh = plsc.VectorSubcoreMesh(core_axis_name="c", subcore_axis_name="s")

    @functools.partial(
        pl.kernel,
        out_type=jax.ShapeDtypeStruct((NC, n_pad, d), jnp.float32),
        mesh=mesh,
        scratch_types=[
            pltpu.VMEM((2, chunk), jnp.int32),
            pltpu.VMEM((2, chunk), jnp.int32),
            pltpu.VMEM((chunk,), jnp.float32),
            pltpu.VMEM((chunk,), jnp.float32),
            pltpu.VMEM((chunk, d), jnp.float32),
            pltpu.VMEM((chunk, d), jnp.float32),
            pltpu.VMEM_SHARED((n_pad, d), jnp.float32),
            pltpu.SemaphoreType.DMA,
            pltpu.SemaphoreType.DMA,
            pltpu.SemaphoreType.DMA,
            pltpu.SemaphoreType.DMA,
        ],
        compiler_params=_sc_compiler_params(),
    )
    def k(h3_hbm, row_hbm, col_hbm, attr_hbm, out_hbm,
          ibuf0, ibuf1, abuf0, abuf1, gbuf0, gbuf1, acc,
          si0, si1, sg0, sg1):
        c = lax.axis_index("c")
        s = lax.axis_index("s")
        wid = s * NC + c
        base = wid * epw

        ibuf = (ibuf0, ibuf1)
        abuf = (abuf0, abuf1)
        gbuf = (gbuf0, gbuf1)
        si = (si0, si1)
        sg = (sg0, sg1)

        def issue_idx(i, b):
            off = base + i * chunk
            pltpu.async_copy(row_hbm.at[pl.ds(off, chunk)], ibuf[b].at[0], si[b])
            pltpu.async_copy(col_hbm.at[pl.ds(off, chunk)], ibuf[b].at[1], si[b])
            pltpu.async_copy(attr_hbm.at[pl.ds(off, chunk)], abuf[b], si[b])

        def wait_idx(i, b):
            off = base + i * chunk
            pltpu.make_async_copy(row_hbm.at[pl.ds(off, chunk)], ibuf[b].at[0], si[b]).wait()
            pltpu.make_async_copy(col_hbm.at[pl.ds(off, chunk)], ibuf[b].at[1], si[b]).wait()
            pltpu.make_async_copy(attr_hbm.at[pl.ds(off, chunk)], abuf[b], si[b]).wait()

        def issue_gather(b):
            pltpu.async_copy(h3_hbm.at[ibuf[b].at[0]], gbuf[b], sg[b])

        def wait_gather(b):
            pltpu.make_async_copy(h3_hbm.at[ibuf[b].at[0]], gbuf[b], sg[b]).wait()

        @pl.loop(0, zrows)
        def _(i):
            for kk in range(d // LANES):
                gbuf0[i, pl.ds(kk * LANES, LANES)] = jnp.zeros((LANES,), jnp.float32)

        @pl.loop(0, 5)
        def _(t):
            pltpu.sync_copy(gbuf0.at[pl.ds(0, zrows)],
                            acc.at[pl.ds(s * rows_per_tile + t * zrows, zrows)])

        plsc.subcore_barrier()

        issue_idx(0, 0)
        issue_idx(1, 1)
        wait_idx(0, 0)
        issue_gather(0)

        @pl.loop(0, nchunk // 2)
        def _(t):
            for b in range(2):
                i = t * 2 + b
                nb = 1 - b

                @pl.when(i + 1 < nchunk)
                def _():
                    wait_idx(i + 1, nb)
                    issue_gather(nb)

                wait_gather(b)

                gb = gbuf[b]
                ab = abuf[b]

                @plsc.parallel_loop(0, chunk, unroll=4)
                def _(j):
                    j16 = jnp.zeros((LANES,), jnp.int32) + j
                    w = plsc.load_gather(ab, [j16])
                    for kk in range(d // LANES):
                        sl = pl.ds(kk * LANES, LANES)
                        gb[j, sl] = gb[j, sl] * w

                pltpu.sync_copy(gb, acc.at[ibuf[b].at[1]], add=True)

                @pl.when(i + 2 < nchunk)
                def _():
                    issue_idx(i + 2, b)

        plsc.subcore_barrier()

        @pl.loop(0, 5)
        def _(t):
            st = s * rows_per_tile + t * zrows
            pltpu.sync_copy(acc.at[pl.ds(st, zrows)],
                            out_hbm.at[c, pl.ds(st, zrows)])

    return k(h3, row, col, attr)


def _bn(x, gamma, beta):
    mean = jnp.mean(x, axis=0)
    xc = x - mean
    var = jnp.mean(xc * xc, axis=0)
    return gamma * xc / jnp.sqrt(var + 1e-5) + beta


def _tc_layer0(x, gamma, beta, w):
    def body(x_ref, g_ref, b_ref, w_ref, o_ref):
        h = _bn(x_ref[...], g_ref[...], b_ref[...])
        o_ref[...] = jnp.dot(h, w_ref[...], preferred_element_type=jnp.float32)

    return pl.pallas_call(
        body,
        out_shape=jax.ShapeDtypeStruct(x.shape, jnp.float32),
    )(x, gamma, beta, w)


def _tc_scale(degp, h2):
    n, d = h2.shape

    def body(degp_ref, h2_ref, dis_ref, h3_ref):
        deg = degp_ref[0, :n, :LANES] + degp_ref[1, :n, :LANES] + 2.0
        dis = jnp.where(deg > 0, lax.rsqrt(deg), 0.0)
        dis_ref[...] = dis
        h3_ref[...] = dis[:, 0:1] * h2_ref[...]

    return pl.pallas_call(
        body,
        out_shape=[
            jax.ShapeDtypeStruct((n, LANES), jnp.float32),
            jax.ShapeDtypeStruct((n, d), jnp.float32),
        ],
    )(degp, h2)


def _tc_combine_next(raw, h2, dis16, b, gamma, beta, w):
    n, d = h2.shape

    def body(raw_ref, h2_ref, dis_ref, b_ref, g_ref, bt_ref, w_ref,
             h2o_ref, h3o_ref):
        d1 = dis_ref[:, 0:1]
        h = (d1 * (raw_ref[0, :n, :] + raw_ref[1, :n, :])
             + (2.0 * d1 * d1) * h2_ref[...] + b_ref[...])
        h = jnp.where(h >= 0, h, 0.01 * h)
        h = _bn(h, g_ref[...], bt_ref[...])
        h2n = jnp.dot(h, w_ref[...], preferred_element_type=jnp.float32)
        h2o_ref[...] = h2n
        h3o_ref[...] = d1 * h2n

    return pl.pallas_call(
        body,
        out_shape=[
            jax.ShapeDtypeStruct((n, d), jnp.float32),
            jax.ShapeDtypeStruct((n, d), jnp.float32),
        ],
    )(raw, h2, dis16, b, gamma, beta, w)


def _tc_finish_pool(raw, h2, dis16, b, batch, g):
    n, d = h2.shape

    def body(raw_ref, h2_ref, dis_ref, b_ref, batch_ref, o_ref):
        d1 = dis_ref[:, 0:1]
        h = (d1 * (raw_ref[0, :n, :] + raw_ref[1, :n, :])
             + (2.0 * d1 * d1) * h2_ref[...] + b_ref[...])
        h = jnp.where(h >= 0, h, 0.01 * h)
        gids = lax.broadcasted_iota(jnp.int32, (g, n), 0)
        oh = (gids == batch_ref[...][None, :]).astype(jnp.float32)
        sums = jnp.dot(oh, h, preferred_element_type=jnp.float32)
        cnt = jnp.sum(oh, axis=1)
        o_ref[...] = sums / jnp.maximum(cnt, 1.0)[:, None]

    return pl.pallas_call(
        body,
        out_shape=jax.ShapeDtypeStruct((g, d), jnp.float32),
    )(raw, h2, dis16, b, batch)


def kernel(x, edge_index, edge_attr, batch,
           bn_gamma0, bn_beta0, W0, b0,
           bn_gamma1, bn_beta1, W1, b1,
           bn_gamma2, bn_beta2, W2, b2):
    n, d = x.shape
    e = edge_index.shape[1]
    g = 64
    n_pad = ((n + NS * 40 - 1) // (NS * 40)) * (NS * 40)
    row = edge_index[0]
    col = edge_index[1]
    ew = 1.0 - edge_attr[:, 0]

    epw0 = e // NW
    nchunk = (epw0 + CHUNK - 1) // CHUNK
    nchunk += nchunk % 2
    padn = nchunk * CHUNK - epw0
    if padn:
        ar = jnp.arange(padn, dtype=jnp.int32)
        row = jnp.concatenate(
            [row.reshape(NW, epw0),
             jnp.broadcast_to((ar * 131) % n, (NW, padn))], axis=1).reshape(-1)
        col = jnp.concatenate(
            [col.reshape(NW, epw0),
             jnp.broadcast_to(n + ar % (n_pad - n), (NW, padn))], axis=1).reshape(-1)
        ew = jnp.concatenate(
            [ew.reshape(NW, epw0),
             jnp.zeros((NW, padn), jnp.float32)], axis=1).reshape(-1)

    degp = _sc_degree(col, ew, n_pad, nchunk)
    h2 = _tc_layer0(x, bn_gamma0, bn_beta0, W0)
    dis16, h3 = _tc_scale(degp, h2)

    params = [(b0, bn_gamma1, bn_beta1, W1), (b1, bn_gamma2, bn_beta2, W2)]
    for (bl, gm, bt, wl) in params:
        raw = _sc_aggregate(h3, row, col, ew, n_pad, d, nchunk)
        h2, h3 = _tc_combine_next(raw, h2, dis16, bl, gm, bt, wl)

    raw = _sc_aggregate(h3, row, col, ew, n_pad, d, nchunk)
    return _tc_finish_pool(raw, h2, dis16, b2, batch, g)
